# baseline (speedup 1.0000x reference)
import functools

import jax
import jax.numpy as jnp
from jax import lax
from jax.experimental import pallas as pl
from jax.experimental.pallas import tpu as pltpu


def _sum_body(x_ref, out_ref, s_s, *, nb):
    j = pl.program_id(0)

    @pl.when(j == 0)
    def _init():
        s_s[...] = jnp.zeros_like(s_s[...])

    s_s[...] += jnp.sum(x_ref[...], keepdims=True)

    @pl.when(j == nb - 1)
    def _fin():
        out_ref[...] = s_s[...]


def kernel(x, target):
    B, C = x.shape
    bm = 32
    nb = B // bm
    body = functools.partial(_sum_body, nb=nb)
    out = pl.pallas_call(
        body,
        grid=(nb,),
        in_specs=[pl.BlockSpec((bm, C), lambda j: (j, 0))],
        out_specs=pl.BlockSpec((1, 1), lambda j: (0, 0)),
        out_shape=jax.ShapeDtypeStruct((1, 1), jnp.float32),
        scratch_shapes=[pltpu.VMEM((1, 1), jnp.float32)],
    )(x)
    return out[0, 0]


# speedup vs baseline: 2.9273x; 2.9273x over previous
import functools

import jax
import jax.numpy as jnp
from jax import lax
from jax.experimental import pallas as pl
from jax.experimental.pallas import tpu as pltpu

_MS = 2.0


def _lse_body(xt_ref, t_ref, out_ref, m_s, s_s, v_s, *, nb, bc, n_cls, n_rows):
    j = pl.program_id(0)

    @pl.when(j == 0)
    def _init():
        m_s[...] = jnp.full_like(m_s[...], -jnp.inf)
        s_s[...] = jnp.zeros_like(s_s[...])
        v_s[...] = jnp.zeros_like(v_s[...])

    def step(xb, cls):
        bm = jnp.max(xb, axis=0, keepdims=True)
        m_old = m_s[...]
        m_new = jnp.maximum(m_old, bm)
        s_s[...] = s_s[...] * jnp.exp(m_old - m_new) + jnp.sum(
            jnp.exp(xb - m_new), axis=0, keepdims=True)
        m_s[...] = m_new
        v_s[...] += jnp.sum(jnp.where(cls == t_ref[...], xb, 0.0), axis=0,
                            keepdims=True)

    cls = j * bc + lax.broadcasted_iota(jnp.int32, xt_ref.shape, 0)

    @pl.when(j < nb - 1)
    def _main():
        step(xt_ref[...], cls)

    @pl.when(j == nb - 1)
    def _last():
        step(jnp.where(cls < n_cls, xt_ref[...], -jnp.inf), cls)
        vm2 = v_s[...] - _MS
        m = m_s[...]
        lse = m + jnp.log(s_s[...] - jnp.exp(vm2 + _MS - m) + jnp.exp(vm2 - m))
        out_ref[...] = jnp.sum(lse - vm2, keepdims=True) / n_rows


def kernel(x, target):
    B, C = x.shape
    xt = x.T
    bc = 2048
    nb = pl.cdiv(C, bc)
    body = functools.partial(_lse_body, nb=nb, bc=bc, n_cls=C, n_rows=B)
    out = pl.pallas_call(
        body,
        grid=(nb,),
        in_specs=[
            pl.BlockSpec((bc, B), lambda j: (j, 0)),
            pl.BlockSpec((1, B), lambda j: (0, 0)),
        ],
        out_specs=pl.BlockSpec((1, 1), lambda j: (0, 0)),
        out_shape=jax.ShapeDtypeStruct((1, 1), jnp.float32),
        scratch_shapes=[
            pltpu.VMEM((1, B), jnp.float32),
            pltpu.VMEM((1, B), jnp.float32),
            pltpu.VMEM((1, B), jnp.float32),
        ],
    )(xt, target.reshape(1, B))
    return out[0, 0]


# speedup vs baseline: 2.9700x; 1.0146x over previous
import functools

import jax
import jax.numpy as jnp
from jax import lax
from jax.experimental import pallas as pl
from jax.experimental.pallas import tpu as pltpu

_MS = 2.0


def _lse_body(xt_ref, t_ref, out_ref, m_s, s_s, v_s, *, nb, bc, n_rows):
    j = pl.program_id(0)

    @pl.when(j == 0)
    def _init():
        m_s[...] = jnp.full_like(m_s[...], -jnp.inf)
        s_s[...] = jnp.zeros_like(s_s[...])
        v_s[...] = jnp.zeros_like(v_s[...])

    def step(xb, cls):
        bm = jnp.max(xb, axis=0, keepdims=True)
        m_old = m_s[...]
        m_new = jnp.maximum(m_old, bm)
        s_s[...] = s_s[...] * jnp.exp(m_old - m_new) + jnp.sum(
            jnp.exp(xb - m_new), axis=0, keepdims=True)
        m_s[...] = m_new
        v_s[...] += jnp.sum(jnp.where(cls == t_ref[...], xb, 0.0), axis=0,
                            keepdims=True)

    cls = j * bc + lax.broadcasted_iota(jnp.int32, xt_ref.shape, 0)
    step(xt_ref[...], cls)

    @pl.when(j == nb - 1)
    def _last():
        vm2 = v_s[...] - _MS
        m = m_s[...]
        lse = m + jnp.log(s_s[...] - jnp.exp(vm2 + _MS - m) + jnp.exp(vm2 - m))
        out_ref[...] = jnp.sum(lse - vm2, keepdims=True) / n_rows


def kernel(x, target):
    B, C = x.shape
    xt = x.T
    bc = 2000
    assert C % bc == 0
    nb = C // bc
    body = functools.partial(_lse_body, nb=nb, bc=bc, n_rows=B)
    out = pl.pallas_call(
        body,
        grid=(nb,),
        in_specs=[
            pl.BlockSpec((bc, B), lambda j: (j, 0)),
            pl.BlockSpec((1, B), lambda j: (0, 0)),
        ],
        out_specs=pl.BlockSpec((1, 1), lambda j: (0, 0)),
        out_shape=jax.ShapeDtypeStruct((1, 1), jnp.float32),
        scratch_shapes=[
            pltpu.VMEM((1, B), jnp.float32),
            pltpu.VMEM((1, B), jnp.float32),
            pltpu.VMEM((1, B), jnp.float32),
        ],
    )(xt, target.reshape(1, B))
    return out[0, 0]


# speedup vs baseline: 3.1408x; 1.0575x over previous
import functools

import jax
import jax.numpy as jnp
from jax import lax
from jax.experimental import pallas as pl
from jax.experimental.pallas import tpu as pltpu

_MS = 2.0


def _lse_body(xt_ref, t_ref, out_ref, m_s, s_s, v_s, *, nb, bc, n_rows):
    j = pl.program_id(0)

    @pl.when(j == 0)
    def _init():
        m_s[...] = jnp.full_like(m_s[...], -jnp.inf)
        s_s[...] = jnp.zeros_like(s_s[...])
        v_s[...] = jnp.zeros_like(v_s[...])

    def step(xb, cls):
        bm = jnp.max(xb, axis=0, keepdims=True)
        m_old = m_s[...]
        m_new = jnp.maximum(m_old, bm)
        s_s[...] = s_s[...] * jnp.exp(m_old - m_new) + jnp.sum(
            jnp.exp(xb - m_new), axis=0, keepdims=True)
        m_s[...] = m_new
        v_s[...] += jnp.sum(jnp.where(cls == t_ref[...], xb, 0.0), axis=0,
                            keepdims=True)

    cls = j * bc + lax.broadcasted_iota(jnp.int32, xt_ref.shape, 0)
    step(xt_ref[...], cls)

    @pl.when(j == nb - 1)
    def _last():
        vm2 = v_s[...] - _MS
        m = m_s[...]
        lse = m + jnp.log(s_s[...] - jnp.exp(vm2 + _MS - m) + jnp.exp(vm2 - m))
        out_ref[...] = jnp.sum(lse - vm2, keepdims=True) / n_rows


def kernel(x, target):
    B, C = x.shape
    xt = x.T
    bc = 4000
    assert C % bc == 0
    nb = C // bc
    body = functools.partial(_lse_body, nb=nb, bc=bc, n_rows=B)
    out = pl.pallas_call(
        body,
        grid=(nb,),
        in_specs=[
            pl.BlockSpec((bc, B), lambda j: (j, 0)),
            pl.BlockSpec((1, B), lambda j: (0, 0)),
        ],
        out_specs=pl.BlockSpec((1, 1), lambda j: (0, 0)),
        out_shape=jax.ShapeDtypeStruct((1, 1), jnp.float32),
        scratch_shapes=[
            pltpu.VMEM((1, B), jnp.float32),
            pltpu.VMEM((1, B), jnp.float32),
            pltpu.VMEM((1, B), jnp.float32),
        ],
    )(xt, target.reshape(1, B))
    return out[0, 0]
